# Initial kernel scaffold; baseline (speedup 1.0000x reference)
import jax
import jax.numpy as jnp
from jax.experimental import pallas as pl


def kernel(bboxes, gt_bboxes, inside_gt_bbox_mask):
    raise NotImplementedError("write your pallas kernel here")


# speedup vs baseline: 2.5331x; 2.5331x over previous
import functools

import jax
import jax.numpy as jnp
from jax import lax
from jax.experimental import pallas as pl
from jax.experimental.pallas import tpu as pltpu
from jax.experimental.pallas import tpu_sc as plsc

N = 20000
G = 64
L = 16
NG = G // L
NC, NS = 2, 16
NW = NC * NS
BLK = 160
NBLK = N // BLK
ITERS = -(-NBLK // NW)
TPB = BLK // L
RATIOS = (1.0, 0.75, 0.5, 0.25)
EPS = 1e-7
LN2 = 0.6931471805599453
SQRT2 = 1.4142135623730951
BIG = 1e30


def _vlog(x):
    xi = plsc.bitcast(x, jnp.int32)
    e = (xi >> 23) - 127
    m = plsc.bitcast((xi & 0x007FFFFF) | 0x3F800000, jnp.float32)
    big = m > SQRT2
    m = jnp.where(big, m * 0.5, m)
    e = e + jnp.where(big, 1, 0)
    z = (m - 1.0) / (m + 1.0)
    z2 = z * z
    p = z * (2.0 + z2 * (2.0 / 3.0 + z2 * (0.4 + z2 * (2.0 / 7.0))))
    return e.astype(jnp.float32) * LN2 + p


def _vsqrt(x):
    xi = plsc.bitcast(x, jnp.int32)
    y = plsc.bitcast((xi >> 1) + 0x1FBD1DF5, jnp.float32)
    for _ in range(3):
        y = 0.5 * (y + x / y)
    return y


def _insert3(t1, t2, t3, v):
    n1 = jnp.minimum(t1, v)
    h1 = jnp.maximum(t1, v)
    n2 = jnp.minimum(t2, h1)
    h2 = jnp.maximum(t2, h1)
    n3 = jnp.minimum(t3, h2)
    return n1, n2, n3


def _feats(x, y, X, Y):
    b0, b1, b2, b3 = x, y, X, Y
    for r in RATIOS:
        cx = (b2 + b0) * 0.5
        cy = (b3 + b1) * 0.5
        w = b2 - b0
        h = b3 - b1
        b0 = cx - w * r * 0.5
        b1 = cy - h * r * 0.5
        b2 = cx + w * r * 0.5
        b3 = cy + h * r * 0.5
    cx1 = (b0 + b2) * 0.5
    cy1 = (b1 + b3) * 0.5
    hx = 0.5 * (b2 - b0)
    hy = 0.5 * (b3 - b1)
    v1x = hx * hx + EPS
    v1y = hy * hy + EPS
    f = _vlog(v1x * v1y)
    return cx1, cy1, v1x, v1y, f


def _bc(v, r):
    return jnp.take_along_axis(v, jnp.full((L,), r, dtype=jnp.int32), axis=0)


def _gt_consts(gtv):
    out = []
    for g in range(NG):
        sl = pl.ds(g * L, L)
        x2, y2, X2, Y2 = gtv[0, sl], gtv[1, sl], gtv[2, sl], gtv[3, sl]
        c = (x2 + X2) * 0.5
        d = (y2 + Y2) * 0.5
        hx = 0.5 * (X2 - x2)
        hy = 0.5 * (Y2 - y2)
        v2x = hx * hx + EPS
        v2y = hy * hy + EPS
        out.append((1.0 / v2x, 1.0 / v2y, c, d, v2x, v2y))
    return out


def _ka_body(bbB, gtT, top3, colv, gtv, tv):
    wid = lax.axis_index("s") * NC + lax.axis_index("c")
    pltpu.sync_copy(gtT, gtv)
    consts = _gt_consts(gtv)

    def block_body(i, T):
        b = wid + NW * i
        active = b < NBLK

        @pl.when(active)
        def _():
            pltpu.sync_copy(bbB.at[b], colv)

        def tile_body(t, T):
            Ts = list(T)
            sl = pl.ds(t * L, L)
            cx, cy, vx, vy, f = _feats(colv[0, sl], colv[1, sl],
                                       colv[2, sl], colv[3, sl])
            for r in range(L):
                cxr, cyr = _bc(cx, r), _bc(cy, r)
                vxr, vyr = _bc(vx, r), _bc(vy, r)
                fr = _bc(f, r)
                for g in range(NG):
                    a, bq, cc, dd, _, _ = consts[g]
                    dxx = cxr - cc
                    P = vxr + dxx * dxx
                    dyy = cyr - dd
                    Q = vyr + dyy * dyy
                    w = P * a + Q * bq - fr
                    w = jnp.where(active, w, BIG)
                    Ts[3 * g], Ts[3 * g + 1], Ts[3 * g + 2] = _insert3(
                        Ts[3 * g], Ts[3 * g + 1], Ts[3 * g + 2], w)
            return tuple(Ts)

        return lax.fori_loop(0, TPB, tile_body, T)

    T0 = tuple(jnp.full((L,), BIG, dtype=jnp.float32) for _ in range(3 * NG))
    T = lax.fori_loop(0, ITERS, block_body, T0)
    for g in range(NG):
        sl = pl.ds(g * L, L)
        for k in range(3):
            tv[k, sl] = T[3 * g + k]
    pltpu.sync_copy(tv, top3.at[wid])


def _kb_body(bbB, gtT, top3, insd, out, colv, gtv, t3v, inv, outv):
    wid = lax.axis_index("s") * NC + lax.axis_index("c")
    pltpu.sync_copy(gtT, gtv)
    pltpu.sync_copy(top3, t3v)
    consts = _gt_consts(gtv)

    params = []
    for g in range(NG):
        a, bq, c, d, v2x, v2y = consts[g]
        e = _vlog(v2x * v2y) - 2.0
        sl = pl.ds(g * L, L)

        m1 = m2 = m3 = jnp.full((L,), BIG, dtype=jnp.float32)
        for wi in range(NW):
            for k in range(3):
                m1, m2, m3 = _insert3(m1, m2, m3, t3v[wi, k, sl])
        ep2 = e + 2.0
        o1 = 2.0 / (m1 + ep2)
        o2 = 2.0 / (m2 + ep2)
        o3 = 2.0 / (m3 + ep2)
        mean = ((o1 + o2) * 4.0 + o3) / 9.0
        d1, d2, d3 = o1 - mean, o2 - mean, o3 - mean
        var = ((d1 * d1 + d2 * d2) * 4.0 + d3 * d3) / 8.0
        var = jnp.maximum(var, 0.0)
        thr = mean + _vsqrt(var)
        Rg = 2.0 / thr - 2.0 - e
        params.append((a, bq, c, d, e, Rg))

    def block_body(i, carry):
        b = wid + NW * i

        @pl.when(b < NBLK)
        def _():
            base = b * BLK
            pltpu.sync_copy(bbB.at[b], colv)
            pltpu.sync_copy(insd.at[pl.ds(base, BLK)], inv)

            def tile_body(t, c2):
                sl = pl.ds(t * L, L)
                cx, cy, vx, vy, f = _feats(colv[0, sl], colv[1, sl],
                                           colv[2, sl], colv[3, sl])
                for r in range(L):
                    rr = t * L + r
                    cxr, cyr = _bc(cx, r), _bc(cy, r)
                    vxr, vyr = _bc(vx, r), _bc(vy, r)
                    fr = _bc(f, r)
                    ws = []
                    mm = None
                    for g in range(NG):
                        a, bq, cc, dd, e, _ = params[g]
                        dxx = cxr - cc
                        P = vxr + dxx * dxx
                        dyy = cyr - dd
                        Q = vyr + dyy * dyy
                        w = P * a + Q * bq
                        u = w + e
                        mm = u if mm is None else jnp.minimum(mm, u)
                        ws.append(w)
                    m2k = jnp.min(mm - fr)
                    fg = (m2k <= 4.0 / 3.0) & (m2k >= 2.0 / 19.0)
                    for g in range(NG):
                        _, _, _, _, _, Rg = params[g]
                        gsl = pl.ds(g * L, L)
                        sel = (ws[g] <= Rg + fr) & fg
                        ob = sel | (inv[rr, gsl] != 0)
                        outv[rr, gsl] = jnp.where(ob, 1, 0)
                return c2

            lax.fori_loop(0, TPB, tile_body, 0)
            pltpu.sync_copy(outv, out.at[pl.ds(base, BLK)])

        return carry

    lax.fori_loop(0, ITERS, block_body, 0)


@functools.cache
def _build():
    mesh = plsc.VectorSubcoreMesh(core_axis_name="c", subcore_axis_name="s",
                                  num_cores=NC, num_subcores=NS)
    params = pltpu.CompilerParams(needs_layout_passes=False,
                                  use_tc_tiling_on_sc=False)
    ka = pl.kernel(
        _ka_body,
        out_type=jax.ShapeDtypeStruct((NW, 3, G), jnp.float32),
        mesh=mesh,
        compiler_params=params,
        scratch_types=[
            pltpu.VMEM((4, BLK), jnp.float32),
            pltpu.VMEM((4, G), jnp.float32),
            pltpu.VMEM((3, G), jnp.float32),
        ],
    )
    kb = pl.kernel(
        _kb_body,
        out_type=jax.ShapeDtypeStruct((N, G), jnp.int32),
        mesh=mesh,
        compiler_params=params,
        scratch_types=[
            pltpu.VMEM((4, BLK), jnp.float32),
            pltpu.VMEM((4, G), jnp.float32),
            pltpu.VMEM((NW, 3, G), jnp.float32),
            pltpu.VMEM((BLK, G), jnp.int32),
            pltpu.VMEM((BLK, G), jnp.int32),
        ],
    )
    return ka, kb


def kernel(bboxes, gt_bboxes, inside_gt_bbox_mask):
    ka, kb = _build()
    bbB = bboxes.T.reshape(4, NBLK, BLK).transpose(1, 0, 2)
    gtT = gt_bboxes.T
    insd = inside_gt_bbox_mask.astype(jnp.int32)
    top3 = ka(bbB, gtT)
    outi = kb(bbB, gtT, top3, insd)
    return outi.astype(bool)


# speedup vs baseline: 2.7332x; 1.0790x over previous
import functools

import jax
import jax.numpy as jnp
from jax import lax
from jax.experimental import pallas as pl
from jax.experimental.pallas import tpu as pltpu
from jax.experimental.pallas import tpu_sc as plsc

N = 20000
G = 64
L = 16
NG = G // L
NC, NS = 2, 16
NW = NC * NS
BLK = 160
NBLK = N // BLK
ITERS = -(-NBLK // NW)
TPB = BLK // L
RATIOS = (1.0, 0.75, 0.5, 0.25)
EPS = 1e-7
LN2 = 0.6931471805599453
SQRT2 = 1.4142135623730951
BIG = 1e30


def _vlog(x):
    xi = plsc.bitcast(x, jnp.int32)
    e = (xi >> 23) - 127
    m = plsc.bitcast((xi & 0x007FFFFF) | 0x3F800000, jnp.float32)
    big = m > SQRT2
    m = jnp.where(big, m * 0.5, m)
    e = e + jnp.where(big, 1, 0)
    z = (m - 1.0) / (m + 1.0)
    z2 = z * z
    p = z * (2.0 + z2 * (2.0 / 3.0 + z2 * (0.4 + z2 * (2.0 / 7.0))))
    return e.astype(jnp.float32) * LN2 + p


def _vsqrt(x):
    xi = plsc.bitcast(x, jnp.int32)
    y = plsc.bitcast((xi >> 1) + 0x1FBD1DF5, jnp.float32)
    for _ in range(3):
        y = 0.5 * (y + x / y)
    return y


def _insert3(t1, t2, t3, v):
    n1 = jnp.minimum(t1, v)
    h1 = jnp.maximum(t1, v)
    n2 = jnp.minimum(t2, h1)
    h2 = jnp.maximum(t2, h1)
    n3 = jnp.minimum(t3, h2)
    return n1, n2, n3


def _feats(x, y, X, Y):
    b0, b1, b2, b3 = x, y, X, Y
    for r in RATIOS:
        cx = (b2 + b0) * 0.5
        cy = (b3 + b1) * 0.5
        w = b2 - b0
        h = b3 - b1
        b0 = cx - w * r * 0.5
        b1 = cy - h * r * 0.5
        b2 = cx + w * r * 0.5
        b3 = cy + h * r * 0.5
    cx1 = (b0 + b2) * 0.5
    cy1 = (b1 + b3) * 0.5
    hx = 0.5 * (b2 - b0)
    hy = 0.5 * (b3 - b1)
    v1x = hx * hx + EPS
    v1y = hy * hy + EPS
    f = _vlog(v1x * v1y)
    return cx1, cy1, v1x, v1y, f


def _bc(v, r):
    return jnp.take_along_axis(v, jnp.full((L,), r, dtype=jnp.int32), axis=0)


def _gt_consts(gtv):
    out = []
    for g in range(NG):
        sl = pl.ds(g * L, L)
        x2, y2, X2, Y2 = gtv[0, sl], gtv[1, sl], gtv[2, sl], gtv[3, sl]
        c = (x2 + X2) * 0.5
        d = (y2 + Y2) * 0.5
        hx = 0.5 * (X2 - x2)
        hy = 0.5 * (Y2 - y2)
        v2x = hx * hx + EPS
        v2y = hy * hy + EPS
        out.append((1.0 / v2x, 1.0 / v2y, c, d, v2x, v2y))
    return out


def _ka_body(bbB, gtT, top3, colv, gtv, tv):
    wid = lax.axis_index("s") * NC + lax.axis_index("c")
    pltpu.sync_copy(gtT, gtv)
    consts = _gt_consts(gtv)

    def block_body(i, T):
        b = wid + NW * i
        active = b < NBLK

        @pl.when(active)
        def _():
            pltpu.sync_copy(bbB.at[b], colv)

        def tile_body(t, T):
            Ts = list(T)
            sl = pl.ds(t * L, L)
            cx, cy, vx, vy, f = _feats(colv[0, sl], colv[1, sl],
                                       colv[2, sl], colv[3, sl])
            for r in range(L):
                cxr, cyr = _bc(cx, r), _bc(cy, r)
                vxr, vyr = _bc(vx, r), _bc(vy, r)
                fr = _bc(f, r)
                for g in range(NG):
                    a, bq, cc, dd, _, _ = consts[g]
                    dxx = cxr - cc
                    P = vxr + dxx * dxx
                    dyy = cyr - dd
                    Q = vyr + dyy * dyy
                    w = P * a + Q * bq - fr
                    w = jnp.where(active, w, BIG)
                    Ts[3 * g], Ts[3 * g + 1], Ts[3 * g + 2] = _insert3(
                        Ts[3 * g], Ts[3 * g + 1], Ts[3 * g + 2], w)
            return tuple(Ts)

        return lax.fori_loop(0, TPB, tile_body, T)

    T0 = tuple(jnp.full((L,), BIG, dtype=jnp.float32) for _ in range(3 * NG))
    T = lax.fori_loop(0, ITERS, block_body, T0)
    for g in range(NG):
        sl = pl.ds(g * L, L)
        for k in range(3):
            tv[k, sl] = T[3 * g + k]
    pltpu.sync_copy(tv, top3.at[wid])


def _kb_body(bbB, gtT, top3, insd, out, colv, gtv, t3v, inv, outv):
    wid = lax.axis_index("s") * NC + lax.axis_index("c")
    pltpu.sync_copy(gtT, gtv)
    pltpu.sync_copy(top3, t3v)
    consts = _gt_consts(gtv)

    params = []
    for g in range(NG):
        a, bq, c, d, v2x, v2y = consts[g]
        e = _vlog(v2x * v2y) - 2.0
        sl = pl.ds(g * L, L)

        m1 = m2 = m3 = jnp.full((L,), BIG, dtype=jnp.float32)
        for wi in range(NW):
            for k in range(3):
                m1, m2, m3 = _insert3(m1, m2, m3, t3v[wi, k, sl])
        ep2 = e + 2.0
        o1 = 2.0 / (m1 + ep2)
        o2 = 2.0 / (m2 + ep2)
        o3 = 2.0 / (m3 + ep2)
        mean = ((o1 + o2) * 4.0 + o3) / 9.0
        d1, d2, d3 = o1 - mean, o2 - mean, o3 - mean
        var = ((d1 * d1 + d2 * d2) * 4.0 + d3 * d3) / 8.0
        var = jnp.maximum(var, 0.0)
        thr = mean + _vsqrt(var)
        Rg = 2.0 / thr - 2.0 - e
        params.append((a, bq, c, d, e, Rg))

    def block_body(i, carry):
        b = wid + NW * i

        @pl.when(b < NBLK)
        def _():
            base = b * BLK
            pltpu.sync_copy(bbB.at[b], colv)
            pltpu.sync_copy(insd.at[pl.ds(base, BLK)], inv)

            def tile_body(t, c2):
                sl = pl.ds(t * L, L)
                cx, cy, vx, vy, f = _feats(colv[0, sl], colv[1, sl],
                                           colv[2, sl], colv[3, sl])
                for r in range(L):
                    rr = t * L + r
                    cxr, cyr = _bc(cx, r), _bc(cy, r)
                    vxr, vyr = _bc(vx, r), _bc(vy, r)
                    fr = _bc(f, r)
                    ws = []
                    mm = None
                    for g in range(NG):
                        a, bq, cc, dd, e, _ = params[g]
                        dxx = cxr - cc
                        P = vxr + dxx * dxx
                        dyy = cyr - dd
                        Q = vyr + dyy * dyy
                        w = P * a + Q * bq
                        u = w + e
                        mm = u if mm is None else jnp.minimum(mm, u)
                        ws.append(w)
                    m2k = jnp.min(mm - fr)
                    fg = (m2k <= 4.0 / 3.0) & (m2k >= 2.0 / 19.0)
                    packed = plsc.bitcast(inv[rr, :], jnp.int32)
                    for g in range(NG):
                        _, _, _, _, _, Rg = params[g]
                        sel = (ws[g] <= Rg + fr) & fg
                        packed = packed | (jnp.where(sel, 1, 0) << (8 * g))
                    outv[rr, :] = plsc.bitcast(packed, jnp.uint8)
                return c2

            lax.fori_loop(0, TPB, tile_body, 0)
            pltpu.sync_copy(outv, out.at[pl.ds(base, BLK)])

        return carry

    lax.fori_loop(0, ITERS, block_body, 0)


@functools.cache
def _build():
    mesh = plsc.VectorSubcoreMesh(core_axis_name="c", subcore_axis_name="s",
                                  num_cores=NC, num_subcores=NS)
    params = pltpu.CompilerParams(needs_layout_passes=False,
                                  use_tc_tiling_on_sc=False)
    ka = pl.kernel(
        _ka_body,
        out_type=jax.ShapeDtypeStruct((NW, 3, G), jnp.float32),
        mesh=mesh,
        compiler_params=params,
        scratch_types=[
            pltpu.VMEM((4, BLK), jnp.float32),
            pltpu.VMEM((4, G), jnp.float32),
            pltpu.VMEM((3, G), jnp.float32),
        ],
    )
    kb = pl.kernel(
        _kb_body,
        out_type=jax.ShapeDtypeStruct((N, G), jnp.uint8),
        mesh=mesh,
        compiler_params=params,
        scratch_types=[
            pltpu.VMEM((4, BLK), jnp.float32),
            pltpu.VMEM((4, G), jnp.float32),
            pltpu.VMEM((NW, 3, G), jnp.float32),
            pltpu.VMEM((BLK, G), jnp.uint8),
            pltpu.VMEM((BLK, G), jnp.uint8),
        ],
    )
    return ka, kb


def kernel(bboxes, gt_bboxes, inside_gt_bbox_mask):
    ka, kb = _build()
    bbB = bboxes.T.reshape(4, NBLK, BLK).transpose(1, 0, 2)
    perm = (jnp.arange(G) % L) * NG + jnp.arange(G) // L
    gtT = gt_bboxes.T[:, perm]
    insd = inside_gt_bbox_mask.view(jnp.uint8)
    top3 = ka(bbB, gtT)
    outu = kb(bbB, gtT, top3, insd)
    return outu.astype(bool)
